# baseline (speedup 1.0000x reference)
import jax
import jax.numpy as jnp
from jax.experimental import pallas as pl
from jax.experimental.pallas import tpu as pltpu

MEM_ROWS = 4096
VOCAB = 32000
EMBED = 128
HOPS = 3
V_TILE = 640
J_STEPS = VOCAB // V_TILE // 2
M_HALF = MEM_ROWS // 2


def _mc_kernel(x_ref, a_ref, c_ref, q_ref, b_ref, mcp_ref, u0p_ref,
               acc_ref, u0_acc_ref):
    j = pl.program_id(1)
    r = pl.program_id(2)
    w = jnp.concatenate([a_ref[...], c_ref[...]], axis=0)
    part = jax.lax.dot_general(
        x_ref[...].astype(jnp.bfloat16), w.astype(jnp.bfloat16),
        (((1,), (1,)), ((), ())),
        preferred_element_type=jnp.float32)
    rows = pl.ds(r * M_HALF, M_HALF)

    @pl.when(j == 0)
    def _():
        acc_ref[rows, :] = part

    @pl.when(j > 0)
    def _():
        acc_ref[rows, :] = acc_ref[rows, :] + part

    @pl.when(r == 0)
    def _():
        u0part = jax.lax.dot_general(
            q_ref[...].astype(jnp.bfloat16), b_ref[...].astype(jnp.bfloat16),
            (((1,), (1,)), ((), ())),
            preferred_element_type=jnp.float32)
        u0_acc_ref[...] = jnp.where(j == 0, u0part,
                                    u0_acc_ref[...] + u0part)

    @pl.when(j == J_STEPS - 1)
    def _():
        mcp_ref[0, rows, :] = acc_ref[rows, :].astype(jnp.bfloat16)

    @pl.when((j == J_STEPS - 1) & (r == 1))
    def _():
        u0p_ref[0] = u0_acc_ref[...]


def _hops_kernel(mcp_ref, u0p_ref, u_ref):
    mc = mcp_ref[0].astype(jnp.float32) + mcp_ref[1].astype(jnp.float32)
    u = u0p_ref[0] + u0p_ref[1]
    m = mc[:, :EMBED]
    c = mc[:, EMBED:]
    for _ in range(HOPS):
        logits = jax.lax.dot_general(
            u, m, (((1,), (1,)), ((), ())),
            preferred_element_type=jnp.float32)
        logits = logits - jnp.max(logits, axis=-1, keepdims=True)
        p = jnp.exp(logits)
        p = p / jnp.sum(p, axis=-1, keepdims=True)
        o = jnp.dot(p, c, preferred_element_type=jnp.float32)
        u = u + o
    u_ref[...] = u


def kernel(memory, query, A, B, C):
    x = memory.reshape(MEM_ROWS, VOCAB)

    mcp, u0p = pl.pallas_call(
        _mc_kernel,
        grid=(2, J_STEPS, 2),
        in_specs=[
            pl.BlockSpec((M_HALF, V_TILE), lambda kv, j, r: (r, kv * J_STEPS + j)),
            pl.BlockSpec((EMBED, V_TILE), lambda kv, j, r: (0, kv * J_STEPS + j)),
            pl.BlockSpec((EMBED, V_TILE), lambda kv, j, r: (0, kv * J_STEPS + j)),
            pl.BlockSpec((1, V_TILE), lambda kv, j, r: (0, kv * J_STEPS + j)),
            pl.BlockSpec((EMBED, V_TILE), lambda kv, j, r: (0, kv * J_STEPS + j)),
        ],
        out_specs=[
            pl.BlockSpec((1, MEM_ROWS, 2 * EMBED), lambda kv, j, r: (kv, 0, 0)),
            pl.BlockSpec((1, 1, EMBED), lambda kv, j, r: (kv, 0, 0)),
        ],
        out_shape=[
            jax.ShapeDtypeStruct((2, MEM_ROWS, 2 * EMBED), jnp.bfloat16),
            jax.ShapeDtypeStruct((2, 1, EMBED), jnp.float32),
        ],
        scratch_shapes=[
            pltpu.VMEM((MEM_ROWS, 2 * EMBED), jnp.float32),
            pltpu.VMEM((1, EMBED), jnp.float32),
        ],
        compiler_params=pltpu.CompilerParams(
            dimension_semantics=("parallel", "arbitrary", "arbitrary"),
            vmem_limit_bytes=60 * 1024 * 1024,
        ),
    )(x, A, C, query, B)

    u = pl.pallas_call(
        _hops_kernel,
        out_shape=jax.ShapeDtypeStruct((1, EMBED), jnp.float32),
        compiler_params=pltpu.CompilerParams(
            vmem_limit_bytes=40 * 1024 * 1024,
        ),
    )(mcp, u0p)
    return u


# speedup vs baseline: 1.2414x; 1.2414x over previous
import jax
import jax.numpy as jnp
from jax.experimental import pallas as pl
from jax.experimental.pallas import tpu as pltpu

MEM_ROWS = 4096
VOCAB = 32000
EMBED = 128
HOPS = 3
V_TILE = 640
J_STEPS = VOCAB // V_TILE // 2
M_HALF = MEM_ROWS // 2


def _mc_kernel(x_ref, a_ref, c_ref, q_ref, b_ref, mcp_ref, u0p_ref,
               acc_ref, u0_acc_ref):
    j = pl.program_id(1)
    w = jnp.concatenate([a_ref[...], c_ref[...]], axis=0)
    part = jax.lax.dot_general(
        x_ref[...].astype(jnp.bfloat16), w.astype(jnp.bfloat16),
        (((1,), (1,)), ((), ())),
        preferred_element_type=jnp.float32)
    u0part = jax.lax.dot_general(
        q_ref[...].astype(jnp.bfloat16), b_ref[...].astype(jnp.bfloat16),
        (((1,), (1,)), ((), ())),
        preferred_element_type=jnp.float32)

    @pl.when(j == 0)
    def _():
        acc_ref[...] = part
        u0_acc_ref[...] = u0part

    @pl.when(j > 0)
    def _():
        acc_ref[...] = acc_ref[...] + part
        u0_acc_ref[...] = u0_acc_ref[...] + u0part

    @pl.when(j == J_STEPS - 1)
    def _():
        mcp_ref[0] = acc_ref[...].astype(jnp.bfloat16)
        u0p_ref[0] = u0_acc_ref[...]


def _hops_kernel(mcp_ref, u0p_ref, u_ref):
    mc = mcp_ref[0].astype(jnp.float32) + mcp_ref[1].astype(jnp.float32)
    u = u0p_ref[0] + u0p_ref[1]
    m = mc[:, :EMBED]
    c = mc[:, EMBED:]
    for _ in range(HOPS):
        logits = jax.lax.dot_general(
            u, m, (((1,), (1,)), ((), ())),
            preferred_element_type=jnp.float32)
        logits = logits - jnp.max(logits, axis=-1, keepdims=True)
        p = jnp.exp(logits)
        p = p / jnp.sum(p, axis=-1, keepdims=True)
        o = jnp.dot(p, c, preferred_element_type=jnp.float32)
        u = u + o
    u_ref[...] = u


def kernel(memory, query, A, B, C):
    x = memory.reshape(MEM_ROWS, VOCAB)

    mcp, u0p = pl.pallas_call(
        _mc_kernel,
        grid=(2, J_STEPS),
        in_specs=[
            pl.BlockSpec((MEM_ROWS, V_TILE), lambda kv, j: (0, kv * J_STEPS + j)),
            pl.BlockSpec((EMBED, V_TILE), lambda kv, j: (0, kv * J_STEPS + j)),
            pl.BlockSpec((EMBED, V_TILE), lambda kv, j: (0, kv * J_STEPS + j)),
            pl.BlockSpec((1, V_TILE), lambda kv, j: (0, kv * J_STEPS + j)),
            pl.BlockSpec((EMBED, V_TILE), lambda kv, j: (0, kv * J_STEPS + j)),
        ],
        out_specs=[
            pl.BlockSpec((1, MEM_ROWS, 2 * EMBED), lambda kv, j: (kv, 0, 0)),
            pl.BlockSpec((1, 1, EMBED), lambda kv, j: (kv, 0, 0)),
        ],
        out_shape=[
            jax.ShapeDtypeStruct((2, MEM_ROWS, 2 * EMBED), jnp.bfloat16),
            jax.ShapeDtypeStruct((2, 1, EMBED), jnp.float32),
        ],
        scratch_shapes=[
            pltpu.VMEM((MEM_ROWS, 2 * EMBED), jnp.float32),
            pltpu.VMEM((1, EMBED), jnp.float32),
        ],
        compiler_params=pltpu.CompilerParams(
            dimension_semantics=("parallel", "arbitrary"),
            vmem_limit_bytes=60 * 1024 * 1024,
        ),
    )(x, A, C, query, B)

    u = pl.pallas_call(
        _hops_kernel,
        out_shape=jax.ShapeDtypeStruct((1, EMBED), jnp.float32),
        compiler_params=pltpu.CompilerParams(
            vmem_limit_bytes=40 * 1024 * 1024,
        ),
    )(mcp, u0p)
    return u
